# Initial kernel scaffold; baseline (speedup 1.0000x reference)
import jax
import jax.numpy as jnp
from jax.experimental import pallas as pl


def kernel(x, edge_index, batch, W1l, b1, W1r, W2l, b2, W2r, Wp, bp, q_weights, Wf, bf):
    raise NotImplementedError("write your pallas kernel here")


# speedup vs baseline: 4.1520x; 4.1520x over previous
import jax
import jax.numpy as jnp
import numpy as np
from jax import lax
from jax.experimental import pallas as pl
from jax.experimental.pallas import tpu as pltpu
from jax.experimental.pallas import tpu_sc as plsc

N = 10000
E = 320000
F = 128
H = 64
NQ = 8
QL = 4
G = 64

NC = 2
NS = 16
NW = NC * NS
EPW = E // NW
CH = 80
NP = 10240
RPW = NP // NS


def _sc_aggregate(feat, src, dst, zeros_f):
    Fd = feat.shape[1]
    mesh = plsc.VectorSubcoreMesh(core_axis_name="c", subcore_axis_name="s")
    out_type = jax.ShapeDtypeStruct((NC, NP, Fd), jnp.float32)
    scratch = [
        pltpu.VMEM((CH,), jnp.int32),
        pltpu.VMEM((CH,), jnp.int32),
        pltpu.VMEM((CH, Fd), jnp.float32),
        pltpu.VMEM_SHARED((NP, Fd), jnp.float32),
        pltpu.SemaphoreType.DMA,
    ]

    def body(feat_h, src_h, dst_h, zf_h, out_h,
             src_v, dst_v, rows_v, agg_sh, sem):
        c = lax.axis_index("c")
        s = lax.axis_index("s")
        wid = s * NC + c
        row0 = pl.multiple_of(s * RPW, 8)
        pltpu.sync_copy(zf_h, agg_sh.at[pl.ds(row0, RPW)])
        plsc.subcore_barrier()

        def step(i, carry):
            base = pl.multiple_of(wid * EPW + i * CH, 8)
            pltpu.sync_copy(src_h.at[pl.ds(base, CH)], src_v)
            pltpu.sync_copy(dst_h.at[pl.ds(base, CH)], dst_v)
            pltpu.async_copy(feat_h.at[src_v], rows_v, sem).wait()
            pltpu.sync_copy(rows_v, agg_sh.at[dst_v], add=True)
            return carry

        lax.fori_loop(0, EPW // CH, step, 0)
        plsc.subcore_barrier()
        pltpu.sync_copy(agg_sh.at[pl.ds(row0, RPW)],
                        out_h.at[c, pl.ds(row0, RPW)])

    return pl.kernel(body, out_type, mesh=mesh, scratch_types=scratch)(
        feat, src, dst, zeros_f)


def _sc_degree(dst, zeros_f, ones):
    mesh = plsc.VectorSubcoreMesh(core_axis_name="c", subcore_axis_name="s")
    out_type = jax.ShapeDtypeStruct((NC, NP, F), jnp.float32)
    scratch = [
        pltpu.VMEM((CH,), jnp.int32),
        pltpu.VMEM((CH, F), jnp.float32),
        pltpu.VMEM_SHARED((NP, F), jnp.float32),
    ]

    def body(dst_h, zf_h, ones_h, out_h, dst_v, ones_v, deg_sh):
        c = lax.axis_index("c")
        s = lax.axis_index("s")
        wid = s * NC + c
        row0 = pl.multiple_of(s * RPW, 8)
        pltpu.sync_copy(zf_h, deg_sh.at[pl.ds(row0, RPW)])
        pltpu.sync_copy(ones_h, ones_v)
        plsc.subcore_barrier()

        def step(i, carry):
            base = pl.multiple_of(wid * EPW + i * CH, 8)
            pltpu.sync_copy(dst_h.at[pl.ds(base, CH)], dst_v)
            pltpu.sync_copy(ones_v, deg_sh.at[dst_v], add=True)
            return carry

        lax.fori_loop(0, EPW // CH, step, 0)
        plsc.subcore_barrier()
        pltpu.sync_copy(deg_sh.at[pl.ds(row0, RPW)],
                        out_h.at[c, pl.ds(row0, RPW)])

    return pl.kernel(body, out_type, mesh=mesh, scratch_types=scratch)(
        dst, zeros_f, ones)


def _sage_dense(parts, degp, feat, Wl, b, Wr, pad_out=False):
    Nn, Fd = feat.shape
    Hh = Wl.shape[1]
    OW = 128 if pad_out else Hh
    BN = 1024

    def body(p_ref, d_ref, x_ref, wl_ref, b_ref, wr_ref, o_ref):
        p = p_ref[0] + p_ref[1]
        d = d_ref[0, :, 0:1] + d_ref[1, :, 0:1]
        mean = p / jnp.maximum(d, 1.0)
        acc = jnp.dot(mean, wl_ref[...], preferred_element_type=jnp.float32)
        acc = acc + jnp.dot(x_ref[...], wr_ref[...],
                            preferred_element_type=jnp.float32)
        res = jnp.maximum(acc + b_ref[...], 0.0)
        if pad_out:
            res = jnp.concatenate(
                [res, jnp.zeros((BN, OW - Hh), jnp.float32)], axis=1)
        o_ref[...] = res

    return pl.pallas_call(
        body,
        grid=(Nn // BN,),
        in_specs=[
            pl.BlockSpec((NC, BN, Fd), lambda i: (0, i, 0)),
            pl.BlockSpec((NC, BN, F), lambda i: (0, i, 0)),
            pl.BlockSpec((BN, Fd), lambda i: (i, 0)),
            pl.BlockSpec((Fd, Hh), lambda i: (0, 0)),
            pl.BlockSpec((1, Hh), lambda i: (0, 0)),
            pl.BlockSpec((Fd, Hh), lambda i: (0, 0)),
        ],
        out_specs=pl.BlockSpec((BN, OW), lambda i: (i, 0)),
        out_shape=jax.ShapeDtypeStruct((Nn, OW), jnp.float32),
        compiler_params=pltpu.CompilerParams(
            dimension_semantics=("parallel",)),
    )(parts, degp, feat, Wl, b.reshape(1, Hh), Wr)


def _pool_qnn(h2, batch_row, Wp, bp, qw, Wf, bf):
    Nn = h2.shape[0]
    D = 2 ** NQ
    hp = lax.Precision.HIGHEST

    def body(h_ref, b_ref, wp_ref, bp_ref, qw_ref, wf_ref, bf_ref, o_ref):
        h = h_ref[...]
        bid = b_ref[...]
        gid = lax.broadcasted_iota(jnp.int32, (G, Nn), 0)
        onehot = (bid == gid).astype(jnp.float32)
        sums = jnp.dot(onehot, h, precision=hp,
                       preferred_element_type=jnp.float32)
        cnts = jnp.sum(onehot, axis=1, keepdims=True)
        pooled = sums / jnp.maximum(cnts, 1.0)
        z = jnp.dot(pooled, wp_ref[...], precision=hp,
                    preferred_element_type=jnp.float32) + bp_ref[...]
        ang = jnp.tanh(z) * jnp.float32(np.pi)
        half = ang * 0.5
        cA = jnp.cos(half)
        sA = jnp.sin(half)

        I = lax.broadcasted_iota(jnp.int32, (D, D), 0)
        J = lax.broadcasted_iota(jnp.int32, (D, D), 1)
        Pm = [(I == (J ^ (1 << (NQ - 1 - q)))).astype(jnp.float32)
              for q in range(NQ)]
        Cm = []
        for q in range(NQ):
            t = (q + 1) % NQ
            mt = 1 << (NQ - 1 - t)
            cb = (J >> (NQ - 1 - q)) & 1
            Cm.append((I == (J ^ (cb * mt))).astype(jnp.float32))

        re = (lax.broadcasted_iota(jnp.int32, (G, D), 1) == 0)
        re = re.astype(jnp.float32)
        im = jnp.zeros((G, D), jnp.float32)

        def rx(re, im, cq, sq, q):
            reP = jnp.dot(re, Pm[q], precision=hp,
                          preferred_element_type=jnp.float32)
            imP = jnp.dot(im, Pm[q], precision=hp,
                          preferred_element_type=jnp.float32)
            return cq * re + sq * imP, cq * im - sq * reP

        for q in range(NQ):
            re, im = rx(re, im, cA[:, q:q + 1], sA[:, q:q + 1], q)
        qwh = qw_ref[...] * 0.5
        cw = jnp.cos(qwh)
        sw = jnp.sin(qwh)
        for l in range(QL):
            for q in range(NQ):
                re, im = rx(re, im, cw[l:l + 1, q:q + 1],
                            sw[l:l + 1, q:q + 1], q)
            for q in range(NQ):
                re = jnp.dot(re, Cm[q], precision=hp,
                             preferred_element_type=jnp.float32)
                im = jnp.dot(im, Cm[q], precision=hp,
                             preferred_element_type=jnp.float32)

        prob = re * re + im * im
        bi = lax.broadcasted_iota(jnp.int32, (D, NQ), 0)
        bq = lax.broadcasted_iota(jnp.int32, (D, NQ), 1)
        bit = (bi >> ((NQ - 1) - bq)) & 1
        zmat = 1.0 - 2.0 * bit.astype(jnp.float32)
        qo = jnp.dot(prob, zmat, precision=hp,
                     preferred_element_type=jnp.float32)
        o_ref[...] = jnp.dot(qo, wf_ref[...], precision=hp,
                             preferred_element_type=jnp.float32) + bf_ref[...]

    return pl.pallas_call(
        body,
        out_shape=jax.ShapeDtypeStruct((G, 2), jnp.float32),
    )(h2, batch_row, Wp, bp.reshape(1, NQ), qw, Wf, bf.reshape(1, 2))


def kernel(x, edge_index, batch, W1l, b1, W1r, W2l, b2, W2r, Wp, bp,
           q_weights, Wf, bf):
    src = edge_index[0]
    dst = edge_index[1]
    xp = jnp.concatenate(
        [x, jnp.zeros((NP - N, F), jnp.float32)], axis=0)
    zeros_f = jnp.zeros((RPW, F), jnp.float32)
    ones = jnp.ones((CH, F), jnp.float32)
    W2l_p = jnp.concatenate([W2l, jnp.zeros((F - H, H), jnp.float32)], 0)
    W2r_p = jnp.concatenate([W2r, jnp.zeros((F - H, H), jnp.float32)], 0)

    degp = _sc_degree(dst, zeros_f, ones)
    agg1p = _sc_aggregate(xp, src, dst, zeros_f)
    h1 = _sage_dense(agg1p, degp, xp, W1l, b1, W1r, pad_out=True)
    agg2p = _sc_aggregate(h1, src, dst, zeros_f)
    h2 = _sage_dense(agg2p, degp, h1, W2l_p, b2, W2r_p)
    return _pool_qnn(h2[:N], batch.reshape(1, N), Wp, bp, q_weights, Wf, bf)


# speedup vs baseline: 7.5811x; 1.8259x over previous
import jax
import jax.numpy as jnp
import numpy as np
from jax import lax
from jax.experimental import pallas as pl
from jax.experimental.pallas import tpu as pltpu
from jax.experimental.pallas import tpu_sc as plsc

N = 10000
E = 320000
F = 128
H = 64
NQ = 8
QL = 4
G = 64

NC = 2
NS = 16
NW = NC * NS
EPW = E // NW
CH = 80
NP = 10240
RPW = NP // NS


def _sc_aggregate(feat, src, dst, zeros_f):
    Fd = feat.shape[1]
    NCHUNK = EPW // CH
    mesh = plsc.VectorSubcoreMesh(core_axis_name="c", subcore_axis_name="s")
    out_type = jax.ShapeDtypeStruct((NC, NP, Fd), jnp.float32)
    scratch = [
        pltpu.VMEM((CH,), jnp.int32),
        pltpu.VMEM((CH,), jnp.int32),
        pltpu.VMEM((CH, Fd), jnp.float32),
        pltpu.VMEM((CH,), jnp.int32),
        pltpu.VMEM((CH,), jnp.int32),
        pltpu.VMEM((CH, Fd), jnp.float32),
        pltpu.VMEM_SHARED((NP, Fd), jnp.float32),
        pltpu.SemaphoreType.DMA,
        pltpu.SemaphoreType.DMA,
    ]

    def body(feat_h, src_h, dst_h, zf_h, out_h,
             srcA, dstA, rowsA, srcB, dstB, rowsB, agg_sh, semA, semB):
        c = lax.axis_index("c")
        s = lax.axis_index("s")
        wid = s * NC + c
        row0 = pl.multiple_of(s * RPW, 8)
        e0 = wid * EPW
        pltpu.sync_copy(zf_h, agg_sh.at[pl.ds(row0, RPW)])
        plsc.subcore_barrier()

        def start(chunk, sv, dv, rv, sem):
            base = pl.multiple_of(e0 + chunk * CH, 8)
            pltpu.sync_copy(src_h.at[pl.ds(base, CH)], sv)
            pltpu.sync_copy(dst_h.at[pl.ds(base, CH)], dv)
            return pltpu.async_copy(feat_h.at[sv], rv, sem)

        start(0, srcA, dstA, rowsA, semA)

        def step(k, carry):
            start(2 * k + 1, srcB, dstB, rowsB, semB)
            pltpu.make_async_copy(feat_h.at[srcA], rowsA, semA).wait()
            pltpu.sync_copy(rowsA, agg_sh.at[dstA], add=True)
            start(2 * k + 2, srcA, dstA, rowsA, semA)
            pltpu.make_async_copy(feat_h.at[srcB], rowsB, semB).wait()
            pltpu.sync_copy(rowsB, agg_sh.at[dstB], add=True)
            return carry

        lax.fori_loop(0, (NCHUNK - 1) // 2, step, 0)
        pltpu.make_async_copy(feat_h.at[srcA], rowsA, semA).wait()
        pltpu.sync_copy(rowsA, agg_sh.at[dstA], add=True)
        plsc.subcore_barrier()
        pltpu.sync_copy(agg_sh.at[pl.ds(row0, RPW)],
                        out_h.at[c, pl.ds(row0, RPW)])

    return pl.kernel(body, out_type, mesh=mesh, scratch_types=scratch)(
        feat, src, dst, zeros_f)


EB = 8000
EC = E // EB
DHI = NP // 128


def _tc_degree(dst3):
    def body(d_ref, o_ref):
        dv = d_ref[0]
        hi = dv >> 7
        lo = dv & 127
        A = (lax.broadcasted_iota(jnp.int32, (DHI, EB), 0) == hi)
        B = (lax.broadcasted_iota(jnp.int32, (128, EB), 0) == lo)
        dd = lax.dot_general(A.astype(jnp.float32), B.astype(jnp.float32),
                             (((1,), (1,)), ((), ())),
                             preferred_element_type=jnp.float32)

        @pl.when(pl.program_id(0) == 0)
        def _():
            o_ref[...] = jnp.zeros((DHI, 128), jnp.float32)

        o_ref[...] += dd

    return pl.pallas_call(
        body,
        grid=(EC,),
        in_specs=[pl.BlockSpec((1, 1, EB), lambda i: (i, 0, 0))],
        out_specs=pl.BlockSpec((DHI, 128), lambda i: (0, 0)),
        out_shape=jax.ShapeDtypeStruct((DHI, 128), jnp.float32),
    )(dst3)


def _sage_dense(parts, deg2d, feat, Wl, b, Wr, pad_out=False):
    Nn, Fd = feat.shape
    Hh = Wl.shape[1]
    OW = 128 if pad_out else Hh
    BN = 1024
    BH = BN // 128

    def body(p_ref, d_ref, x_ref, wl_ref, b_ref, wr_ref, o_ref):
        p = p_ref[0] + p_ref[1]
        jr = lax.broadcasted_iota(jnp.int32, (BN, BH), 0) >> 7
        P1 = (jr == lax.broadcasted_iota(jnp.int32, (BN, BH), 1))
        X = jnp.dot(P1.astype(jnp.float32), d_ref[...],
                    preferred_element_type=jnp.float32)
        jj = lax.broadcasted_iota(jnp.int32, (BN, 128), 0) & 127
        sel = (jj == lax.broadcasted_iota(jnp.int32, (BN, 128), 1))
        d = jnp.sum(X * sel.astype(jnp.float32), axis=1, keepdims=True)
        mean = p / jnp.maximum(d, 1.0)
        acc = jnp.dot(mean, wl_ref[...], preferred_element_type=jnp.float32)
        acc = acc + jnp.dot(x_ref[...], wr_ref[...],
                            preferred_element_type=jnp.float32)
        res = jnp.maximum(acc + b_ref[...], 0.0)
        if pad_out:
            res = jnp.concatenate(
                [res, jnp.zeros((BN, OW - Hh), jnp.float32)], axis=1)
        o_ref[...] = res

    return pl.pallas_call(
        body,
        grid=(Nn // BN,),
        in_specs=[
            pl.BlockSpec((NC, BN, Fd), lambda i: (0, i, 0)),
            pl.BlockSpec((BH, 128), lambda i: (i, 0)),
            pl.BlockSpec((BN, Fd), lambda i: (i, 0)),
            pl.BlockSpec((Fd, Hh), lambda i: (0, 0)),
            pl.BlockSpec((1, Hh), lambda i: (0, 0)),
            pl.BlockSpec((Fd, Hh), lambda i: (0, 0)),
        ],
        out_specs=pl.BlockSpec((BN, OW), lambda i: (i, 0)),
        out_shape=jax.ShapeDtypeStruct((Nn, OW), jnp.float32),
        compiler_params=pltpu.CompilerParams(
            dimension_semantics=("parallel",)),
    )(parts, deg2d, feat, Wl, b.reshape(1, Hh), Wr)


def _pool_qnn(h2, batch_row, Wp, bp, qw, Wf, bf):
    Nn = h2.shape[0]
    D = 2 ** NQ
    hp = lax.Precision.HIGHEST

    def body(h_ref, b_ref, wp_ref, bp_ref, qw_ref, wf_ref, bf_ref, o_ref):
        h = h_ref[...]
        bid = b_ref[...]
        gid = lax.broadcasted_iota(jnp.int32, (G, Nn), 0)
        onehot = (bid == gid).astype(jnp.float32)
        sums = jnp.dot(onehot, h, precision=hp,
                       preferred_element_type=jnp.float32)
        cnts = jnp.sum(onehot, axis=1, keepdims=True)
        pooled = sums / jnp.maximum(cnts, 1.0)
        z = jnp.dot(pooled, wp_ref[...], precision=hp,
                    preferred_element_type=jnp.float32) + bp_ref[...]
        ang = jnp.tanh(z) * jnp.float32(np.pi)
        half = ang * 0.5
        cA = jnp.cos(half)
        sA = jnp.sin(half)

        I = lax.broadcasted_iota(jnp.int32, (D, D), 0)
        J = lax.broadcasted_iota(jnp.int32, (D, D), 1)
        Pm = [(I == (J ^ (1 << (NQ - 1 - q)))).astype(jnp.float32)
              for q in range(NQ)]
        Cm = []
        for q in range(NQ):
            t = (q + 1) % NQ
            mt = 1 << (NQ - 1 - t)
            cb = (J >> (NQ - 1 - q)) & 1
            Cm.append((I == (J ^ (cb * mt))).astype(jnp.float32))

        re = (lax.broadcasted_iota(jnp.int32, (G, D), 1) == 0)
        re = re.astype(jnp.float32)
        im = jnp.zeros((G, D), jnp.float32)

        def rx(re, im, cq, sq, q):
            reP = jnp.dot(re, Pm[q], precision=hp,
                          preferred_element_type=jnp.float32)
            imP = jnp.dot(im, Pm[q], precision=hp,
                          preferred_element_type=jnp.float32)
            return cq * re + sq * imP, cq * im - sq * reP

        for q in range(NQ):
            re, im = rx(re, im, cA[:, q:q + 1], sA[:, q:q + 1], q)
        qwh = qw_ref[...] * 0.5
        cw = jnp.cos(qwh)
        sw = jnp.sin(qwh)
        for l in range(QL):
            for q in range(NQ):
                re, im = rx(re, im, cw[l:l + 1, q:q + 1],
                            sw[l:l + 1, q:q + 1], q)
            for q in range(NQ):
                re = jnp.dot(re, Cm[q], precision=hp,
                             preferred_element_type=jnp.float32)
                im = jnp.dot(im, Cm[q], precision=hp,
                             preferred_element_type=jnp.float32)

        prob = re * re + im * im
        bi = lax.broadcasted_iota(jnp.int32, (D, NQ), 0)
        bq = lax.broadcasted_iota(jnp.int32, (D, NQ), 1)
        bit = (bi >> ((NQ - 1) - bq)) & 1
        zmat = 1.0 - 2.0 * bit.astype(jnp.float32)
        qo = jnp.dot(prob, zmat, precision=hp,
                     preferred_element_type=jnp.float32)
        o_ref[...] = jnp.dot(qo, wf_ref[...], precision=hp,
                             preferred_element_type=jnp.float32) + bf_ref[...]

    return pl.pallas_call(
        body,
        out_shape=jax.ShapeDtypeStruct((G, 2), jnp.float32),
    )(h2, batch_row, Wp, bp.reshape(1, NQ), qw, Wf, bf.reshape(1, 2))


def kernel(x, edge_index, batch, W1l, b1, W1r, W2l, b2, W2r, Wp, bp,
           q_weights, Wf, bf):
    src = edge_index[0]
    dst = edge_index[1]
    xp = jnp.concatenate(
        [x, jnp.zeros((NP - N, F), jnp.float32)], axis=0)
    zeros_f = jnp.zeros((RPW, F), jnp.float32)
    W2l_p = jnp.concatenate([W2l, jnp.zeros((F - H, H), jnp.float32)], 0)
    W2r_p = jnp.concatenate([W2r, jnp.zeros((F - H, H), jnp.float32)], 0)

    deg2d = _tc_degree(dst.reshape(EC, 1, EB))
    agg1p = _sc_aggregate(xp, src, dst, zeros_f)
    h1 = _sage_dense(agg1p, deg2d, xp, W1l, b1, W1r, pad_out=True)
    agg2p = _sc_aggregate(h1, src, dst, zeros_f)
    h2 = _sage_dense(agg2p, deg2d, h1, W2l_p, b2, W2r_p)
    return _pool_qnn(h2[:N], batch.reshape(1, N), Wp, bp, q_weights, Wf, bf)
